# baseline (speedup 1.0000x reference)
import functools

import jax
import jax.numpy as jnp
from jax import lax
from jax.experimental import pallas as pl
from jax.experimental.pallas import tpu as pltpu
from jax.experimental.pallas import tpu_sc as plsc

B, N, F = 64, 4096, 128
L = 16
NC, NS = 2, 16
NW = NC * NS
NB_SC = 32
BPW = NB_SC // NW
CHUNK = 256
NCH = N // CHUNK
TOT = BPW * NCH
NV = F // L
U = 4
CT = 512


def _sc_segment_max(x):
    mesh = plsc.VectorSubcoreMesh(core_axis_name="c", subcore_axis_name="s")

    @functools.partial(
        pl.kernel,
        mesh=mesh,
        out_type=jax.ShapeDtypeStruct((NB_SC, F), jnp.float32),
        scratch_types=[
            pltpu.VMEM((2, CHUNK, F), jnp.float32),
            pltpu.VMEM((BPW, F), jnp.float32),
            pltpu.SemaphoreType.DMA,
            pltpu.SemaphoreType.DMA,
        ],
    )
    def k(x_hbm, out_hbm, buf, acc, sem0, sem1):
        sems = (sem0, sem1)
        wid = lax.axis_index("s") * NC + lax.axis_index("c")
        base = wid * BPW

        def start(j):
            bi, c = divmod(j, NCH)
            slot = j % 2
            return pltpu.async_copy(
                x_hbm.at[base + bi, pl.ds(c * CHUNK, CHUNK)],
                buf.at[slot], sems[slot])

        cps = {0: start(0)}
        for bi in range(BPW):
            accs = tuple(jnp.full((L,), -jnp.inf, jnp.float32)
                         for _ in range(NV))
            for c in range(NCH):
                j = bi * NCH + c
                if j + 1 < TOT:
                    cps[j + 1] = start(j + 1)
                cps.pop(j).wait()
                slot = j % 2

                def row_body(r, a, slot=slot):
                    out = []
                    for f in range(NV):
                        m = a[f]
                        for u in range(U):
                            m = jnp.maximum(
                                m, buf[slot, r * U + u, pl.ds(L * f, L)])
                        out.append(m)
                    return tuple(out)

                accs = lax.fori_loop(0, CHUNK // U, row_body, accs)
            for f in range(NV):
                acc[bi, pl.ds(L * f, L)] = accs[f]
        pltpu.sync_copy(acc, out_hbm.at[pl.ds(base, BPW)])

    return k(x)


def _tc_segment_max(x, nb_sc):
    nb_tc = B - nb_sc

    H = N // 2

    def body(a_ref, b_ref, o_ref):
        o_ref[0, 0] = jnp.maximum(jnp.max(a_ref[0], axis=0),
                                  jnp.max(b_ref[0], axis=0))

    out = pl.pallas_call(
        body,
        grid=(nb_tc,),
        in_specs=[
            pl.BlockSpec((1, H, F), lambda i: (i + nb_sc, 0, 0)),
            pl.BlockSpec((1, H, F), lambda i: (i + nb_sc, 1, 0)),
        ],
        out_specs=pl.BlockSpec((1, 1, F), lambda i: (i, 0, 0)),
        out_shape=jax.ShapeDtypeStruct((nb_tc, 1, F), jnp.float32),
    )(x, x)
    return out.reshape(nb_tc, F)


def kernel(x):
    return _tc_segment_max(x, 0)


# speedup vs baseline: 1.0453x; 1.0453x over previous
import functools

import jax
import jax.numpy as jnp
from jax import lax
from jax.experimental import pallas as pl
from jax.experimental.pallas import tpu as pltpu
from jax.experimental.pallas import tpu_sc as plsc

B, N, F = 64, 4096, 128
L = 16
NC, NS = 2, 16
NW = NC * NS
NB_SC = 32
BPW = NB_SC // NW
CHUNK = 256
NCH = N // CHUNK
TOT = BPW * NCH
NV = F // L
U = 4
CT = 512


def _sc_segment_max(x):
    mesh = plsc.VectorSubcoreMesh(core_axis_name="c", subcore_axis_name="s")

    @functools.partial(
        pl.kernel,
        mesh=mesh,
        out_type=jax.ShapeDtypeStruct((NB_SC, F), jnp.float32),
        scratch_types=[
            pltpu.VMEM((2, CHUNK, F), jnp.float32),
            pltpu.VMEM((BPW, F), jnp.float32),
            pltpu.SemaphoreType.DMA,
            pltpu.SemaphoreType.DMA,
        ],
    )
    def k(x_hbm, out_hbm, buf, acc, sem0, sem1):
        sems = (sem0, sem1)
        wid = lax.axis_index("s") * NC + lax.axis_index("c")
        base = wid * BPW

        def start(j):
            bi, c = divmod(j, NCH)
            slot = j % 2
            return pltpu.async_copy(
                x_hbm.at[base + bi, pl.ds(c * CHUNK, CHUNK)],
                buf.at[slot], sems[slot])

        cps = {0: start(0)}
        for bi in range(BPW):
            accs = tuple(jnp.full((L,), -jnp.inf, jnp.float32)
                         for _ in range(NV))
            for c in range(NCH):
                j = bi * NCH + c
                if j + 1 < TOT:
                    cps[j + 1] = start(j + 1)
                cps.pop(j).wait()
                slot = j % 2

                def row_body(r, a, slot=slot):
                    out = []
                    for f in range(NV):
                        m = a[f]
                        for u in range(U):
                            m = jnp.maximum(
                                m, buf[slot, r * U + u, pl.ds(L * f, L)])
                        out.append(m)
                    return tuple(out)

                accs = lax.fori_loop(0, CHUNK // U, row_body, accs)
            for f in range(NV):
                acc[bi, pl.ds(L * f, L)] = accs[f]
        pltpu.sync_copy(acc, out_hbm.at[pl.ds(base, BPW)])

    return k(x)


def _tc_segment_max(x, nb_sc):
    nb_tc = B - nb_sc

    NSTR = 4
    H = N // NSTR

    def body(*refs):
        o_ref = refs[-1]
        m = jnp.max(refs[0][0], axis=0)
        for r in refs[1:-1]:
            m = jnp.maximum(m, jnp.max(r[0], axis=0))
        o_ref[0, 0] = m

    out = pl.pallas_call(
        body,
        grid=(nb_tc,),
        in_specs=[
            pl.BlockSpec((1, H, F), lambda i, s=s: (i + nb_sc, s, 0))
            for s in range(NSTR)
        ],
        out_specs=pl.BlockSpec((1, 1, F), lambda i: (i, 0, 0)),
        out_shape=jax.ShapeDtypeStruct((nb_tc, 1, F), jnp.float32),
    )(*([x] * NSTR))
    return out.reshape(nb_tc, F)


def kernel(x):
    return _tc_segment_max(x, 0)
